# Initial kernel scaffold; baseline (speedup 1.0000x reference)
import jax
import jax.numpy as jnp
from jax.experimental import pallas as pl


def kernel(node_embeddings, knn_indices, W1, b1, W2, b2):
    raise NotImplementedError("write your pallas kernel here")


# speedup vs baseline: 13.8203x; 13.8203x over previous
import functools

import jax
import jax.numpy as jnp
from jax import lax
from jax.experimental import pallas as pl
from jax.experimental.pallas import tpu as pltpu
from jax.experimental.pallas import tpu_sc as plsc

B, N, K, D = 2, 10000, 16, 128
NS = 16
NB = 624
NBL = N - 15 * NB
G = 8
RG = G * K
CH = NBL // G
LANES = 16


def _sc_neighbor_sum(emb2d, idx_flat):
    mesh = plsc.VectorSubcoreMesh(core_axis_name="c", subcore_axis_name="s")

    @functools.partial(
        pl.kernel,
        out_type=jax.ShapeDtypeStruct((B * N, D), jnp.float32),
        mesh=mesh,
        scratch_types=[
            pltpu.VMEM((NBL * K,), jnp.int32),
            pltpu.VMEM((RG, D), jnp.float32),
            pltpu.VMEM((NBL, D), jnp.float32),
            pltpu.SemaphoreType.DMA,
        ],
    )
    def ksum(emb_hbm, idx_hbm, out_hbm, idx_v, gbuf, sums, sem):
        c = lax.axis_index("c")
        s = lax.axis_index("s")
        node0 = c * N + s * NB
        pltpu.sync_copy(idx_hbm.at[pl.ds(node0 * K, NBL * K)], idx_v)

        def chunk(t, carry):
            pltpu.async_copy(
                emb_hbm.at[idx_v.at[pl.ds(t * RG, RG)]], gbuf, sem).wait()
            for g in range(G):
                for d8 in range(D // LANES):
                    col = pl.ds(d8 * LANES, LANES)
                    acc = gbuf[g * K, col]
                    for k in range(1, K):
                        acc = acc + gbuf[g * K + k, col]
                    sums[t * G + g, col] = acc
            return carry

        lax.fori_loop(0, CH, chunk, 0)
        pltpu.sync_copy(sums.at[pl.ds(0, NB)], out_hbm.at[pl.ds(node0, NB)])

        @pl.when(s == NS - 1)
        def _():
            pltpu.sync_copy(sums.at[pl.ds(NB, NBL - NB)],
                            out_hbm.at[pl.ds(node0 + NB, NBL - NB)])

    return ksum(emb2d, idx_flat)


def _mlp_body(x_ref, s_ref, w1_ref, b1_ref, w2_ref, b2_ref, o_ref):
    x = x_ref[...]
    m = s_ref[...] * (1.0 / K)
    w1 = w1_ref[...]
    wa = w1[:, :D] + w1[:, 2 * D:]
    wb = w1[:, D:2 * D] - w1[:, 2 * D:]
    h = lax.dot_general(x, wa, (((1,), (1,)), ((), ())),
                        preferred_element_type=jnp.float32)
    h = h + lax.dot_general(m, wb, (((1,), (1,)), ((), ())),
                            preferred_element_type=jnp.float32)
    h = h + b1_ref[...]
    h = 0.5 * h * (1.0 + lax.erf(h * (2.0 ** -0.5)))
    u = lax.dot_general(h, w2_ref[...], (((1,), (1,)), ((), ())),
                        preferred_element_type=jnp.float32)
    o_ref[...] = x + u + b2_ref[...]


def _mlp(x2, s2, W1, b1, W2, b2):
    M = B * N
    BM = 2000
    grid = (M // BM,)
    return pl.pallas_call(
        _mlp_body,
        grid=grid,
        in_specs=[
            pl.BlockSpec((BM, D), lambda i: (i, 0)),
            pl.BlockSpec((BM, D), lambda i: (i, 0)),
            pl.BlockSpec((D, 3 * D), lambda i: (0, 0)),
            pl.BlockSpec((1, D), lambda i: (0, 0)),
            pl.BlockSpec((D, D), lambda i: (0, 0)),
            pl.BlockSpec((1, D), lambda i: (0, 0)),
        ],
        out_specs=pl.BlockSpec((BM, D), lambda i: (i, 0)),
        out_shape=jax.ShapeDtypeStruct((M, D), jnp.float32),
    )(x2, s2, W1, b1.reshape(1, D), W2, b2.reshape(1, D))


def kernel(node_embeddings, knn_indices, W1, b1, W2, b2):
    x2 = node_embeddings.reshape(B * N, D)
    idx = knn_indices.astype(jnp.int32)
    idx = idx + (jnp.arange(B, dtype=jnp.int32) * N)[:, None, None]
    s2 = _sc_neighbor_sum(x2, idx.reshape(B * N * K))
    out2 = _mlp(x2, s2, W1, b1, W2, b2)
    return out2.reshape(B, N, D)


# speedup vs baseline: 17.1680x; 1.2422x over previous
import functools

import jax
import jax.numpy as jnp
from jax import lax
from jax.experimental import pallas as pl
from jax.experimental.pallas import tpu as pltpu
from jax.experimental.pallas import tpu_sc as plsc

B, N, K, D = 2, 10000, 16, 128
NS = 16
NB = 624
NBL = N - 15 * NB
G = 8
RG = G * K
CH = NBL // G
LANES = 16


def _sc_neighbor_sum(emb2d, idx_flat):
    mesh = plsc.VectorSubcoreMesh(core_axis_name="c", subcore_axis_name="s")

    @functools.partial(
        pl.kernel,
        out_type=jax.ShapeDtypeStruct((B * N, D), jnp.float32),
        mesh=mesh,
        scratch_types=[
            pltpu.VMEM((NBL * K,), jnp.int32),
            pltpu.VMEM((RG, D), jnp.float32),
            pltpu.VMEM((RG, D), jnp.float32),
            pltpu.VMEM((NBL, D), jnp.float32),
            pltpu.SemaphoreType.DMA,
            pltpu.SemaphoreType.DMA,
        ],
    )
    def ksum(emb_hbm, idx_hbm, out_hbm, idx_v, gbuf0, gbuf1, sums, sem0,
             sem1):
        c = lax.axis_index("c")
        s = lax.axis_index("s")
        node0 = c * N + s * NB
        pltpu.sync_copy(idx_hbm.at[pl.ds(node0 * K, NBL * K)], idx_v)

        def gather(t, buf, sem):
            return pltpu.async_copy(
                emb_hbm.at[idx_v.at[pl.ds(t * RG, RG)]], buf, sem)

        def reduce_chunk(t, buf):
            for g in range(G):
                for d8 in range(D // LANES):
                    col = pl.ds(d8 * LANES, LANES)
                    acc = buf[g * K, col]
                    for k in range(1, K):
                        acc = acc + buf[g * K + k, col]
                    sums[t * G + g, col] = acc

        gather(0, gbuf0, sem0)

        def chunk2(u, carry):
            t0 = u * 2
            pltpu.make_async_copy(
                emb_hbm.at[idx_v.at[pl.ds(t0 * RG, RG)]], gbuf0, sem0).wait()
            gather(t0 + 1, gbuf1, sem1)
            reduce_chunk(t0, gbuf0)
            pltpu.make_async_copy(
                emb_hbm.at[idx_v.at[pl.ds((t0 + 1) * RG, RG)]], gbuf1,
                sem1).wait()

            @pl.when(u < CH // 2 - 1)
            def _():
                gather(t0 + 2, gbuf0, sem0)

            reduce_chunk(t0 + 1, gbuf1)
            return carry

        lax.fori_loop(0, CH // 2, chunk2, 0)
        pltpu.sync_copy(sums.at[pl.ds(0, NB)], out_hbm.at[pl.ds(node0, NB)])

        @pl.when(s == NS - 1)
        def _():
            pltpu.sync_copy(sums.at[pl.ds(NB, NBL - NB)],
                            out_hbm.at[pl.ds(node0 + NB, NBL - NB)])

    return ksum(emb2d, idx_flat)


def _mlp_body(x_ref, s_ref, w1_ref, b1_ref, w2_ref, b2_ref, o_ref):
    x = x_ref[...]
    m = s_ref[...] * (1.0 / K)
    w1 = w1_ref[...]
    wa = w1[:, :D] + w1[:, 2 * D:]
    wb = w1[:, D:2 * D] - w1[:, 2 * D:]
    h = lax.dot_general(x, wa, (((1,), (1,)), ((), ())),
                        preferred_element_type=jnp.float32)
    h = h + lax.dot_general(m, wb, (((1,), (1,)), ((), ())),
                            preferred_element_type=jnp.float32)
    h = h + b1_ref[...]
    h = 0.5 * h * (1.0 + lax.erf(h * (2.0 ** -0.5)))
    u = lax.dot_general(h, w2_ref[...], (((1,), (1,)), ((), ())),
                        preferred_element_type=jnp.float32)
    o_ref[...] = x + u + b2_ref[...]


def _mlp(x2, s2, W1, b1, W2, b2):
    M = B * N
    BM = 2000
    grid = (M // BM,)
    return pl.pallas_call(
        _mlp_body,
        grid=grid,
        in_specs=[
            pl.BlockSpec((BM, D), lambda i: (i, 0)),
            pl.BlockSpec((BM, D), lambda i: (i, 0)),
            pl.BlockSpec((D, 3 * D), lambda i: (0, 0)),
            pl.BlockSpec((1, D), lambda i: (0, 0)),
            pl.BlockSpec((D, D), lambda i: (0, 0)),
            pl.BlockSpec((1, D), lambda i: (0, 0)),
        ],
        out_specs=pl.BlockSpec((BM, D), lambda i: (i, 0)),
        out_shape=jax.ShapeDtypeStruct((M, D), jnp.float32),
    )(x2, s2, W1, b1.reshape(1, D), W2, b2.reshape(1, D))


def kernel(node_embeddings, knn_indices, W1, b1, W2, b2):
    x2 = node_embeddings.reshape(B * N, D)
    idx = knn_indices.astype(jnp.int32)
    idx = idx + (jnp.arange(B, dtype=jnp.int32) * N)[:, None, None]
    s2 = _sc_neighbor_sum(x2, idx.reshape(B * N * K))
    out2 = _mlp(x2, s2, W1, b1, W2, b2)
    return out2.reshape(B, N, D)
